# baseline (speedup 1.0000x reference)
import functools

import jax
import jax.numpy as jnp
from jax import lax
from jax.experimental import pallas as pl
from jax.experimental.pallas import tpu as pltpu
from jax.experimental.pallas import tpu_sc as plsc

_DT = 7
_NW = 32
_L = 16


def _dense_body(xlr_ref, xan_ref, ylr_ref, msk_ref, dy_ref, ga_ref):
    xlr = xlr_ref[...]
    dy_ref[...] = (ylr_ref[...] - xlr) * msk_ref[...]
    ga_ref[...] = xlr + xan_ref[...]


def _dense_stage(x, ylr, msk_lr):
    B, T2, H, W = x.shape
    T = T2 // 2
    grid = (B * T,)
    bs = (1, 1, H, W)
    lr_spec = pl.BlockSpec(bs, lambda i: (i // T, i % T, 0, 0))
    an_spec = pl.BlockSpec(bs, lambda i: (i // T, T + i % T, 0, 0))
    o_spec = pl.BlockSpec(bs, lambda i: (i // T, i % T, 0, 0))
    out_shape = [
        jax.ShapeDtypeStruct((B, T, H, W), jnp.float32),
        jax.ShapeDtypeStruct((B, T, H, W), jnp.float32),
    ]
    return pl.pallas_call(
        _dense_body,
        grid=grid,
        in_specs=[lr_spec, an_spec, o_spec, o_spec],
        out_specs=[o_spec, o_spec],
        out_shape=out_shape,
    )(x, x, ylr, msk_lr)


def _interp_stage(wid, table_ref, st_h, sy_h, sx_h, sv_h, out_h,
                  bufs, sems, pbuf, T, H, W, C, cpb, nch, tsc_row):
    cst, csy, csx, csv, wbuf, tbuf, idxbuf, gbuf, obuf = bufs
    sem_c, sem_v, sem_g, sem_o = sems
    NG = C // _L
    RW = table_ref.shape[1]
    nb = table_ref.shape[0] // (H * W)
    gt0 = pbuf[0]
    tsc = pbuf[tsc_row]
    gy0 = pbuf[2]
    ysc = pbuf[3]
    gx0 = pbuf[4]
    xsc = pbuf[5]
    lane_i = lax.iota(jnp.int32, _L)
    count = nch // _NW
    assert count >= 2 and count % 2 == 0

    def base_of(k):
        return (wid + k * _NW) * C

    def fire_cxy(k, b):
        base = base_of(k)
        pltpu.async_copy(st_h.at[pl.ds(base, C)], cst[b], sem_c[b])
        pltpu.async_copy(sy_h.at[pl.ds(base, C)], csy[b], sem_c[b])
        pltpu.async_copy(sx_h.at[pl.ds(base, C)], csx[b], sem_c[b])

    def fire_sv(k, b):
        pltpu.async_copy(sv_h.at[pl.ds(base_of(k), C)], csv[b], sem_v[b])

    def do_idx(k, b):
        pltpu.make_async_copy(st_h.at[pl.ds(0, C)], cst[b], sem_c[b]).wait()
        pltpu.make_async_copy(sy_h.at[pl.ds(0, C)], csy[b], sem_c[b]).wait()
        pltpu.make_async_copy(sx_h.at[pl.ds(0, C)], csx[b], sem_c[b]).wait()
        cid = wid + k * _NW
        rb = jnp.minimum(cid // cpb, nb - 1) * (H * W)

        def idx_pass(g, _):
            o = g * _L
            ti = (cst[b][pl.ds(o, _L)] - gt0) * tsc
            yi = (csy[b][pl.ds(o, _L)] - gy0) * ysc
            xi = (csx[b][pl.ds(o, _L)] - gx0) * xsc
            valid = ((ti >= 0.0) & (ti <= T - 1.0)
                     & (yi >= 0.0) & (yi <= H - 1.0)
                     & (xi >= 0.0) & (xi <= W - 1.0))
            t0 = jnp.clip(ti, 0.0, T - 2.0).astype(jnp.int32)
            y0 = jnp.clip(yi, 0.0, H - 2.0).astype(jnp.int32)
            x0 = jnp.clip(xi, 0.0, W - 2.0).astype(jnp.int32)
            wt = jnp.clip(ti - t0.astype(jnp.float32), 0.0, 1.0)
            wy = jnp.clip(yi - y0.astype(jnp.float32), 0.0, 1.0)
            wx = jnp.clip(xi - x0.astype(jnp.float32), 0.0, 1.0)
            col00 = rb + y0 * W + x0
            ib = idxbuf[b]
            ib[pl.ds(0 * C + o, _L)] = col00
            ib[pl.ds(1 * C + o, _L)] = col00 + 1
            ib[pl.ds(2 * C + o, _L)] = col00 + W
            ib[pl.ds(3 * C + o, _L)] = col00 + (W + 1)
            tbuf[b][pl.ds(o, _L)] = t0
            wb = wbuf[b]
            wb[pl.ds(0 * C + o, _L)] = wt
            wb[pl.ds(1 * C + o, _L)] = wy
            wb[pl.ds(2 * C + o, _L)] = wx
            wb[pl.ds(3 * C + o, _L)] = jnp.where(valid, 1.0, 0.0)
            return 0

        lax.fori_loop(0, NG, idx_pass, 0)
        pltpu.async_copy(table_ref.at[idxbuf[b]], gbuf[b], sem_g[b])

    def do_mix(k, b, wait_out):
        pltpu.make_async_copy(table_ref.at[idxbuf[b]], gbuf[b],
                              sem_g[b]).wait()
        pltpu.make_async_copy(sv_h.at[pl.ds(0, C)], csv[b], sem_v[b]).wait()
        if wait_out is not None:
            def _w():
                pltpu.make_async_copy(
                    obuf[b], out_h.at[pl.ds(0, C)], sem_o[b]).wait()
            if wait_out is True:
                _w()
            else:
                pl.when(wait_out)(_w)

        def mix_pass(g, _):
            o = g * _L
            wb = wbuf[b]
            gb = gbuf[b]
            wt = wb[pl.ds(0 * C + o, _L)]
            wy = wb[pl.ds(1 * C + o, _L)]
            wx = wb[pl.ds(2 * C + o, _L)]
            vld = wb[pl.ds(3 * C + o, _L)]
            t0 = tbuf[b][pl.ds(o, _L)]
            t1 = t0 + 1
            r0 = o + lane_i
            cs = []
            for q in range(4):
                rq = r0 + q * C
                v0 = plsc.load_gather(gb, [rq, t0])
                v1 = plsc.load_gather(gb, [rq, t1])
                cs.append(v0 * (1.0 - wt) + v1 * wt)
            c0 = cs[0] * (1.0 - wx) + cs[1] * wx
            c1 = cs[2] * (1.0 - wx) + cs[3] * wx
            sx_val = c0 * (1.0 - wy) + c1 * wy
            obuf[b][pl.ds(o, _L)] = (sx_val - csv[b][pl.ds(o, _L)]) * vld
            return 0

        lax.fori_loop(0, NG, mix_pass, 0)
        pltpu.async_copy(obuf[b], out_h.at[pl.ds(base_of(k), C)], sem_o[b])

    fire_cxy(0, 0)
    fire_sv(0, 0)
    fire_cxy(1, 1)
    fire_sv(1, 1)
    do_idx(0, 0)

    if count > 2:
        def pair(j, _):
            k = 2 * j
            fire_cxy(k + 2, 0)
            do_idx(k + 1, 1)
            do_mix(k, 0, wait_out=(j > 0))
            fire_sv(k + 2, 0)
            fire_cxy(k + 3, 1)
            do_idx(k + 2, 0)
            do_mix(k + 1, 1, wait_out=(j > 0))
            fire_sv(k + 3, 1)
            return 0

        lax.fori_loop(0, (count - 2) // 2, pair, 0)

    tail_wait = True if count > 2 else None
    do_idx(count - 1, 1)
    do_mix(count - 2, 0, wait_out=tail_wait)
    do_mix(count - 1, 1, wait_out=tail_wait)
    pltpu.make_async_copy(obuf[0], out_h.at[pl.ds(0, C)], sem_o[0]).wait()
    pltpu.make_async_copy(obuf[1], out_h.at[pl.ds(0, C)], sem_o[1]).wait()


_NBUF = 9


def _make_sc_kernel(B, T2, H, W, NSP, NNP, CS, CN, cpb_s, cpb_n, RS, RN):
    T = T2 // 2

    mesh = plsc.VectorSubcoreMesh(core_axis_name="c", subcore_axis_name="s")

    def _stage_bufs(C, RW):
        return ([pltpu.VMEM((C,), jnp.float32) for _ in range(2)]
                + [pltpu.VMEM((C,), jnp.float32) for _ in range(2)]
                + [pltpu.VMEM((C,), jnp.float32) for _ in range(2)]
                + [pltpu.VMEM((C,), jnp.float32) for _ in range(2)]
                + [pltpu.VMEM((4 * C,), jnp.float32) for _ in range(2)]
                + [pltpu.VMEM((C,), jnp.int32) for _ in range(2)]
                + [pltpu.VMEM((4 * C,), jnp.int32) for _ in range(2)]
                + [pltpu.VMEM((4 * C, RW), jnp.float32) for _ in range(2)]
                + [pltpu.VMEM((C,), jnp.float32) for _ in range(2)])

    @functools.partial(
        pl.kernel,
        out_type=[
            jax.ShapeDtypeStruct((NSP,), jnp.float32),
            jax.ShapeDtypeStruct((NNP,), jnp.float32),
        ],
        mesh=mesh,
        compiler_params=pltpu.CompilerParams(
            needs_layout_passes=False, use_tc_tiling_on_sc=False),
        scratch_types=(
            _stage_bufs(CS, RS) + _stage_bufs(CN, RN)
            + [pltpu.VMEM((8, _L), jnp.float32)]
            + [pltpu.SemaphoreType.DMA for _ in range(8)]
        ),
    )
    def sc_kernel(tableA_h, tableX_h, st_h, sy_h, sx_h, sv_h,
                  nt_h, ny_h, nx_h, nv_h, params_h,
                  dyout_h, dyout1_h, *scr):
        nsb = 2 * _NBUF
        sbufs = [(scr[2 * i], scr[2 * i + 1]) for i in range(_NBUF)]
        nbufs = [(scr[nsb + 2 * i], scr[nsb + 2 * i + 1])
                 for i in range(_NBUF)]
        pbuf = scr[2 * nsb]
        sems = [(scr[2 * nsb + 1 + 2 * i], scr[2 * nsb + 2 + 2 * i])
                for i in range(4)]
        wid = lax.axis_index("s") * 2 + lax.axis_index("c")
        pltpu.sync_copy(params_h, pbuf)
        _interp_stage(wid, tableA_h, st_h, sy_h, sx_h, sv_h, dyout_h,
                      sbufs, sems, pbuf, T, H, W, CS, cpb_s,
                      NSP // CS, 1)
        _interp_stage(wid, tableX_h, nt_h, ny_h, nx_h, nv_h, dyout1_h,
                      nbufs, sems, pbuf, T2, H, W, CN, cpb_n,
                      NNP // CN, 6)

    return sc_kernel


def _pad_to(a, n):
    return jnp.pad(a.reshape(-1), (0, n - a.size))


def kernel(x, ylr, msk_lr, gt, gy, gx, st, sy, sx, sv, nt, ny, nx, nv):
    B, T2, H, W = x.shape
    T = T2 // 2
    _, NT, NXs = st.shape
    NN = nt.shape[1]
    NS = B * NT * NXs
    NNF = B * NN

    dyoutlr, gridA = _dense_stage(x, ylr, msk_lr)

    tden = gt[-1] - gt[0]
    params = jnp.stack([
        gt[0], (T - 1.0) / tden, gy[0], 1.0 / (gy[1] - gy[0]),
        gx[0], 1.0 / (gx[1] - gx[0]), (T2 - 1.0) / tden, 0.0 * gt[0],
    ]).astype(jnp.float32)
    params = jnp.broadcast_to(params[:, None], (8, _L))

    CS, CN = 320, 400
    ppb_s = NT * NXs
    assert ppb_s % CS == 0
    nch_s = -(-NS // CS)
    nch_s += (-nch_s) % (2 * _NW)
    NSP = nch_s * CS
    nch_n = -(-NNF // CN)
    nch_n += (-nch_n) % (2 * _NW)
    NNP = nch_n * CN

    RS, RN = 8, 16
    tabA = jnp.concatenate(
        [jnp.moveaxis(gridA, 1, -1),
         jnp.zeros((B, H, W, RS - T), jnp.float32)], -1).reshape(-1, RS)
    tabX = jnp.concatenate(
        [jnp.moveaxis(x, 1, -1),
         jnp.zeros((B, H, W, RN - T2), jnp.float32)], -1).reshape(-1, RN)

    sc = _make_sc_kernel(B, T2, H, W, NSP, NNP, CS, CN,
                         ppb_s // CS, NN // CN, RS, RN)
    dyout_flat, dyout1_flat = sc(
        tabA, tabX,
        _pad_to(st, NSP), _pad_to(sy, NSP), _pad_to(sx, NSP),
        _pad_to(sv, NSP),
        _pad_to(nt, NNP), _pad_to(ny, NNP), _pad_to(nx, NNP),
        _pad_to(nv, NNP),
        params)

    return (dyoutlr,
            dyout_flat[:NS].reshape(B, NT, NXs),
            dyout1_flat[:NNF].reshape(B, NN))
